# baseline (speedup 1.0000x reference)
import functools

import jax
import jax.numpy as jnp
from jax import lax
from jax.experimental import pallas as pl
from jax.experimental.pallas import tpu as pltpu
from jax.experimental.pallas import tpu_sc as plsc

N_TOKENS = 8192
HIDDEN = 2048
PRED_H = 256
N_EXPERTS = 64
N_CAND = 16
BT = 512
NB = N_TOKENS // BT

NW = 32
TPW = N_TOKENS // NW
NEG = jnp.float32(-1e9)


def _tdot(a, b):
    return jax.lax.dot_general(a, b, (((1,), (1,)), ((), ())),
                               preferred_element_type=jnp.float32)


def _mm_body(x_ref, w1_ref, b1_ref, w2_ref, b2_ref, eb_ref, rw_ref,
             p_out_ref, f_out_ref):
    x = x_ref[...]
    h = jnp.maximum(_tdot(x, w1_ref[...]) + b1_ref[...], 0.0)
    p_out_ref[...] = _tdot(h, w2_ref[...]) + (b2_ref[...] + eb_ref[...])
    f_out_ref[...] = _tdot(x, rw_ref[...])


def _logits(x, fc1_w, b1, fc2_w, b2, eb, router_w):
    return pl.pallas_call(
        _mm_body,
        grid=(NB,),
        in_specs=[
            pl.BlockSpec((BT, HIDDEN), lambda i: (i, 0)),
            pl.BlockSpec((PRED_H, HIDDEN), lambda i: (0, 0)),
            pl.BlockSpec((1, PRED_H), lambda i: (0, 0)),
            pl.BlockSpec((N_EXPERTS, PRED_H), lambda i: (0, 0)),
            pl.BlockSpec((1, N_EXPERTS), lambda i: (0, 0)),
            pl.BlockSpec((1, N_EXPERTS), lambda i: (0, 0)),
            pl.BlockSpec((N_EXPERTS, HIDDEN), lambda i: (0, 0)),
        ],
        out_specs=[
            pl.BlockSpec((BT, N_EXPERTS), lambda i: (i, 0)),
            pl.BlockSpec((BT, N_EXPERTS), lambda i: (i, 0)),
        ],
        out_shape=[
            jax.ShapeDtypeStruct((N_TOKENS, N_EXPERTS), jnp.float32),
            jax.ShapeDtypeStruct((N_TOKENS, N_EXPERTS), jnp.float32),
        ],
    )(x, fc1_w, b1, fc2_w, b2, eb, router_w)


def _sort16(v):
    s, _ = plsc.sort_key_val(v, v)
    return s


@functools.partial(
    pl.kernel,
    mesh=plsc.VectorSubcoreMesh(core_axis_name="c", subcore_axis_name="s"),
    compiler_params=pltpu.CompilerParams(needs_layout_passes=False),
    out_type=[
        jax.ShapeDtypeStruct((2, N_TOKENS), jnp.float32),
        jax.ShapeDtypeStruct((2, N_TOKENS), jnp.int32),
    ],
    scratch_types=[
        pltpu.VMEM((TPW, N_EXPERTS), jnp.float32),
        pltpu.VMEM((TPW, N_EXPERTS), jnp.float32),
        pltpu.VMEM((TPW,), jnp.float32),
        pltpu.VMEM((TPW,), jnp.float32),
        pltpu.VMEM((TPW,), jnp.int32),
        pltpu.VMEM((TPW,), jnp.int32),
    ],
)
def _sc_tail(p_hbm, f_hbm, w_hbm, id_hbm,
             p_v, f_v, w1_v, w2_v, i1_v, i2_v):
    wid = lax.axis_index("s") * 2 + lax.axis_index("c")
    base = wid * TPW
    pltpu.sync_copy(p_hbm.at[pl.ds(base, TPW)], p_v)
    pltpu.sync_copy(f_hbm.at[pl.ds(base, TPW)], f_v)

    iota = lax.iota(jnp.int32, 16)
    zf = jnp.zeros((16,), jnp.float32)
    zi = jnp.zeros((16,), jnp.int32)

    def tok(t):
        p0 = p_v[t, 0:16]
        p1 = p_v[t, 16:32]
        p2 = p_v[t, 32:48]
        p3 = p_v[t, 48:64]
        s01 = _sort16(jnp.maximum(_sort16(p0), lax.rev(_sort16(p1), (0,))))
        s23 = _sort16(jnp.maximum(_sort16(p2), lax.rev(_sort16(p3), (0,))))
        t16 = jnp.min(jnp.maximum(s01, lax.rev(s23, (0,))))

        f0 = f_v[t, 0:16]
        f1 = f_v[t, 16:32]
        f2 = f_v[t, 32:48]
        f3 = f_v[t, 48:64]
        g0 = jnp.where(p0 >= t16, f0, NEG)
        g1 = jnp.where(p1 >= t16, f1, NEG)
        g2 = jnp.where(p2 >= t16, f2, NEG)
        g3 = jnp.where(p3 >= t16, f3, NEG)

        def top1(a0, a1, a2, a3):
            v01 = jnp.maximum(a0, a1)
            x01 = jnp.where(a0 >= a1, iota, iota + 16)
            v23 = jnp.maximum(a2, a3)
            x23 = jnp.where(a2 >= a3, iota + 32, iota + 48)
            v = jnp.maximum(v01, v23)
            x = jnp.where(v01 >= v23, x01, x23)
            vbest = jnp.max(v)
            ibest = jnp.min(jnp.where(v == vbest, x, N_EXPERTS))
            return vbest, ibest

        v1, i1 = top1(g0, g1, g2, g3)
        g0b = jnp.where(iota == i1, NEG, g0)
        g1b = jnp.where(iota + 16 == i1, NEG, g1)
        g2b = jnp.where(iota + 32 == i1, NEG, g2)
        g3b = jnp.where(iota + 48 == i1, NEG, g3)
        v2, i2 = top1(g0b, g1b, g2b, g3b)
        return v1, v2, i1, i2

    def grp_body(g, carry):
        av1, av2, ai1, ai2 = zf, zf, zi, zi
        t0 = g * 16
        for k in range(16):
            v1, v2, i1, i2 = tok(t0 + k)
            lane = iota == k
            av1 = jnp.where(lane, v1, av1)
            av2 = jnp.where(lane, v2, av2)
            ai1 = jnp.where(lane, i1, ai1)
            ai2 = jnp.where(lane, i2, ai2)
        sl = pl.ds(t0, 16)
        e = jnp.exp(av2 - av1)
        inv = 1.0 / (1.0 + e)
        w1_v[sl] = inv
        w2_v[sl] = e * inv
        i1_v[sl] = ai1
        i2_v[sl] = ai2
        return carry

    lax.fori_loop(0, TPW // 16, grp_body, 0)

    pltpu.sync_copy(w1_v, w_hbm.at[0, pl.ds(base, TPW)])
    pltpu.sync_copy(w2_v, w_hbm.at[1, pl.ds(base, TPW)])
    pltpu.sync_copy(i1_v, id_hbm.at[0, pl.ds(base, TPW)])
    pltpu.sync_copy(i2_v, id_hbm.at[1, pl.ds(base, TPW)])


def kernel(x, fc1_w, fc1_b, fc2_w, fc2_b, expert_bias, router_w):
    b1 = fc1_b.reshape(1, PRED_H)
    b2 = fc2_b.reshape(1, N_EXPERTS)
    eb = expert_bias.reshape(1, N_EXPERTS)
    p, f = _logits(x, fc1_w, b1, fc2_w, b2, eb, router_w)
    w_t, id_t = _sc_tail(p, f)
    return w_t.T, id_t.T


# speedup vs baseline: 1.4443x; 1.4443x over previous
import jax
import jax.numpy as jnp
from jax.experimental import pallas as pl
from jax.experimental.pallas import tpu as pltpu

N_TOKENS = 8192
HIDDEN = 2048
PRED_H = 256
N_EXPERTS = 64
N_CAND = 16
BT = 1024
CHUNK = 128
NB = N_TOKENS // BT


def _tdot(a, b):
    return jax.lax.dot_general(a, b, (((1,), (1,)), ((), ())),
                               preferred_element_type=jnp.float32)


def _router_body(x_ref, w1_ref, b1_ref, w2_ref, b2_ref, eb_ref, rw_ref,
                 w_out_ref, id_out_ref, p_scr, f_scr):
    i = pl.program_id(0)

    tslot = jax.lax.rem(i + 1, 2)
    neg = jnp.float32(-1e9)
    cur = p_scr[tslot]
    for _ in range(N_CAND - 1):
        m = jnp.max(cur, axis=1, keepdims=True)
        cur = jnp.where(cur >= m, -jnp.inf, cur)
    thresh = jnp.max(cur, axis=1, keepdims=True)

    f = f_scr[tslot]
    cand = jnp.logical_or(cur == -jnp.inf, cur >= thresh)
    g = jnp.where(cand, f, neg)
    iota = jax.lax.broadcasted_iota(jnp.int32, g.shape, 1)
    v1 = jnp.max(g, axis=1, keepdims=True)
    i1 = jnp.min(jnp.where(g >= v1, iota, N_EXPERTS), axis=1,
                 keepdims=True)
    g2 = jnp.where(iota == i1, neg, g)
    v2 = jnp.max(g2, axis=1, keepdims=True)
    i2 = jnp.min(jnp.where(g2 >= v2, iota, N_EXPERTS), axis=1,
                 keepdims=True)

    e = jnp.exp(v2 - v1)
    inv = 1.0 / (1.0 + e)
    zf = jnp.zeros_like(g)
    zi = jnp.zeros_like(iota)
    w_out_ref[...] = jnp.where(iota == 0, inv,
                               jnp.where(iota == 1, e * inv, zf))
    id_out_ref[...] = jnp.where(iota == 0, i1,
                                jnp.where(iota == 1, i2, zi))

    x = x_ref[...]
    h = jnp.maximum(_tdot(x, w1_ref[...]) + b1_ref[...], 0.0)
    slot = jax.lax.rem(i, 2)
    p_scr[slot] = _tdot(h, w2_ref[...]) + (b2_ref[...] + eb_ref[...])
    f_scr[slot] = _tdot(x, rw_ref[...])


def kernel(x, fc1_w, fc1_b, fc2_w, fc2_b, expert_bias, router_w):
    b1 = fc1_b.reshape(1, PRED_H)
    b2 = fc2_b.reshape(1, N_EXPERTS)
    eb = expert_bias.reshape(1, N_EXPERTS)

    out_w, out_id = pl.pallas_call(
        _router_body,
        grid=(NB + 1,),
        in_specs=[
            pl.BlockSpec((BT, HIDDEN), lambda i: (jnp.minimum(i, NB - 1), 0)),
            pl.BlockSpec((PRED_H, HIDDEN), lambda i: (0, 0)),
            pl.BlockSpec((1, PRED_H), lambda i: (0, 0)),
            pl.BlockSpec((N_EXPERTS, PRED_H), lambda i: (0, 0)),
            pl.BlockSpec((1, N_EXPERTS), lambda i: (0, 0)),
            pl.BlockSpec((1, N_EXPERTS), lambda i: (0, 0)),
            pl.BlockSpec((N_EXPERTS, HIDDEN), lambda i: (0, 0)),
        ],
        out_specs=[
            pl.BlockSpec((BT, N_EXPERTS), lambda i: (jnp.maximum(i - 1, 0), 0)),
            pl.BlockSpec((BT, N_EXPERTS), lambda i: (jnp.maximum(i - 1, 0), 0)),
        ],
        out_shape=[
            jax.ShapeDtypeStruct((N_TOKENS, N_EXPERTS), jnp.float32),
            jax.ShapeDtypeStruct((N_TOKENS, N_EXPERTS), jnp.int32),
        ],
        scratch_shapes=[
            pltpu.VMEM((2, BT, N_EXPERTS), jnp.float32),
            pltpu.VMEM((2, BT, N_EXPERTS), jnp.float32),
        ],
    )(x, fc1_w, b1, fc2_w, b2, eb, router_w)
    return out_w[:, :2], out_id[:, :2]


# speedup vs baseline: 1.4940x; 1.0345x over previous
import jax
import jax.numpy as jnp
from jax.experimental import pallas as pl
from jax.experimental.pallas import tpu as pltpu

N_TOKENS = 8192
HIDDEN = 2048
PRED_H = 256
N_EXPERTS = 64
N_CAND = 16
BT = 512
CHUNK = 128
NB = N_TOKENS // BT


def _tdot(a, b):
    return jax.lax.dot_general(a, b, (((1,), (1,)), ((), ())),
                               preferred_element_type=jnp.float32)


def _router_body(x_ref, w1_ref, b1_ref, w2_ref, b2_ref, eb_ref, rw_ref,
                 w_out_ref, id_out_ref, p_scr, f_scr):
    i = pl.program_id(0)

    tslot = jax.lax.rem(i + 1, 2)
    neg = jnp.float32(-1e9)
    cur = p_scr[tslot]
    for _ in range(N_CAND - 1):
        m = jnp.max(cur, axis=1, keepdims=True)
        cur = jnp.where(cur >= m, -jnp.inf, cur)
    thresh = jnp.max(cur, axis=1, keepdims=True)

    f = f_scr[tslot]
    cand = jnp.logical_or(cur == -jnp.inf, cur >= thresh)
    g = jnp.where(cand, f, neg)
    iota = jax.lax.broadcasted_iota(jnp.int32, g.shape, 1)
    v1 = jnp.max(g, axis=1, keepdims=True)
    i1 = jnp.min(jnp.where(g >= v1, iota, N_EXPERTS), axis=1,
                 keepdims=True)
    g2 = jnp.where(iota == i1, neg, g)
    v2 = jnp.max(g2, axis=1, keepdims=True)
    i2 = jnp.min(jnp.where(g2 >= v2, iota, N_EXPERTS), axis=1,
                 keepdims=True)

    e = jnp.exp(v2 - v1)
    inv = 1.0 / (1.0 + e)
    zf = jnp.zeros_like(g)
    zi = jnp.zeros_like(iota)
    w_out_ref[...] = jnp.where(iota == 0, inv,
                               jnp.where(iota == 1, e * inv, zf))
    id_out_ref[...] = jnp.where(iota == 0, i1,
                                jnp.where(iota == 1, i2, zi))

    x = x_ref[...]
    h = jnp.maximum(_tdot(x, w1_ref[...]) + b1_ref[...], 0.0)
    slot = jax.lax.rem(i, 2)
    p_scr[slot] = _tdot(h, w2_ref[...]) + (b2_ref[...] + eb_ref[...])
    f_scr[slot] = _tdot(x, rw_ref[...])


def kernel(x, fc1_w, fc1_b, fc2_w, fc2_b, expert_bias, router_w):
    b1 = fc1_b.reshape(1, PRED_H)
    b2 = fc2_b.reshape(1, N_EXPERTS)
    eb = expert_bias.reshape(1, N_EXPERTS)

    out_w, out_id = pl.pallas_call(
        _router_body,
        grid=(NB + 1,),
        in_specs=[
            pl.BlockSpec((BT, HIDDEN), lambda i: (jnp.minimum(i, NB - 1), 0)),
            pl.BlockSpec((PRED_H, HIDDEN), lambda i: (0, 0)),
            pl.BlockSpec((1, PRED_H), lambda i: (0, 0)),
            pl.BlockSpec((N_EXPERTS, PRED_H), lambda i: (0, 0)),
            pl.BlockSpec((1, N_EXPERTS), lambda i: (0, 0)),
            pl.BlockSpec((1, N_EXPERTS), lambda i: (0, 0)),
            pl.BlockSpec((N_EXPERTS, HIDDEN), lambda i: (0, 0)),
        ],
        out_specs=[
            pl.BlockSpec((BT, N_EXPERTS), lambda i: (jnp.maximum(i - 1, 0), 0)),
            pl.BlockSpec((BT, N_EXPERTS), lambda i: (jnp.maximum(i - 1, 0), 0)),
        ],
        out_shape=[
            jax.ShapeDtypeStruct((N_TOKENS, N_EXPERTS), jnp.float32),
            jax.ShapeDtypeStruct((N_TOKENS, N_EXPERTS), jnp.int32),
        ],
        scratch_shapes=[
            pltpu.VMEM((2, BT, N_EXPERTS), jnp.float32),
            pltpu.VMEM((2, BT, N_EXPERTS), jnp.float32),
        ],
    )(x, fc1_w, b1, fc2_w, b2, eb, router_w)
    return out_w[:, :2], out_id[:, :2]


# speedup vs baseline: 1.4989x; 1.0032x over previous
import jax
import jax.numpy as jnp
from jax.experimental import pallas as pl
from jax.experimental.pallas import tpu as pltpu

N_TOKENS = 8192
HIDDEN = 2048
PRED_H = 256
N_EXPERTS = 64
N_CAND = 16
BT = 512
CHUNK = 128
NB = N_TOKENS // BT


def _tdot(a, b):
    return jax.lax.dot_general(a, b, (((1,), (1,)), ((), ())),
                               preferred_element_type=jnp.float32)


def _router_body(x_ref, w1_ref, b1_ref, w2_ref, b2_ref, eb_ref, rw_ref,
                 w_out_ref, id_out_ref, p_scr, f_scr):
    i = pl.program_id(0)

    tslot = jax.lax.rem(i + 1, 2)
    neg = jnp.float32(-1e9)
    cur = p_scr[tslot]
    for _ in range(N_CAND - 1):
        m = jnp.max(cur, axis=1, keepdims=True)
        cur = jnp.where(cur >= m, -jnp.inf, cur)
    thresh = jnp.max(cur, axis=1, keepdims=True)

    f = f_scr[tslot]
    cand = jnp.logical_or(cur == -jnp.inf, cur >= thresh)
    g = jnp.where(cand, f, neg)
    iota = jax.lax.broadcasted_iota(jnp.int32, g.shape, 1)
    iota_f = iota.astype(jnp.float32)
    big = jnp.float32(N_EXPERTS)
    v1 = jnp.max(g, axis=1, keepdims=True)
    i1f = jnp.min(jnp.where(g >= v1, iota_f, big), axis=1, keepdims=True)
    g2 = jnp.where(iota_f == i1f, neg, g)
    v2 = jnp.max(g2, axis=1, keepdims=True)
    i2f = jnp.min(jnp.where(g2 >= v2, iota_f, big), axis=1, keepdims=True)

    e = jnp.exp(v2 - v1)
    inv = 1.0 / (1.0 + e)
    zf = jnp.zeros_like(g)
    zi = jnp.zeros_like(iota)
    w_out_ref[...] = jnp.where(iota == 0, inv,
                               jnp.where(iota == 1, e * inv, zf))
    id_out_ref[...] = jnp.where(iota == 0, i1f.astype(jnp.int32),
                                jnp.where(iota == 1, i2f.astype(jnp.int32),
                                          zi))

    x = x_ref[...]
    h = jnp.maximum(_tdot(x, w1_ref[...]) + b1_ref[...], 0.0)
    slot = jax.lax.rem(i, 2)
    p_scr[slot] = _tdot(h, w2_ref[...]) + (b2_ref[...] + eb_ref[...])
    f_scr[slot] = _tdot(x, rw_ref[...])


def kernel(x, fc1_w, fc1_b, fc2_w, fc2_b, expert_bias, router_w):
    b1 = fc1_b.reshape(1, PRED_H)
    b2 = fc2_b.reshape(1, N_EXPERTS)
    eb = expert_bias.reshape(1, N_EXPERTS)

    out_w, out_id = pl.pallas_call(
        _router_body,
        grid=(NB + 1,),
        in_specs=[
            pl.BlockSpec((BT, HIDDEN), lambda i: (jnp.minimum(i, NB - 1), 0)),
            pl.BlockSpec((PRED_H, HIDDEN), lambda i: (0, 0)),
            pl.BlockSpec((1, PRED_H), lambda i: (0, 0)),
            pl.BlockSpec((N_EXPERTS, PRED_H), lambda i: (0, 0)),
            pl.BlockSpec((1, N_EXPERTS), lambda i: (0, 0)),
            pl.BlockSpec((1, N_EXPERTS), lambda i: (0, 0)),
            pl.BlockSpec((N_EXPERTS, HIDDEN), lambda i: (0, 0)),
        ],
        out_specs=[
            pl.BlockSpec((BT, N_EXPERTS), lambda i: (jnp.maximum(i - 1, 0), 0)),
            pl.BlockSpec((BT, N_EXPERTS), lambda i: (jnp.maximum(i - 1, 0), 0)),
        ],
        out_shape=[
            jax.ShapeDtypeStruct((N_TOKENS, N_EXPERTS), jnp.float32),
            jax.ShapeDtypeStruct((N_TOKENS, N_EXPERTS), jnp.int32),
        ],
        scratch_shapes=[
            pltpu.VMEM((2, BT, N_EXPERTS), jnp.float32),
            pltpu.VMEM((2, BT, N_EXPERTS), jnp.float32),
        ],
    )(x, fc1_w, b1, fc2_w, b2, eb, router_w)
    return out_w[:, :2], out_id[:, :2]
